# baseline (speedup 1.0000x reference)
import functools

import jax
import jax.numpy as jnp
from jax import lax
from jax.experimental import pallas as pl
from jax.experimental.pallas import tpu as pltpu
from jax.experimental.pallas import tpu_sc as plsc

B = 16384
D = 64
DP = 128
V = 100001
H = 65536
NC = 2
NS = 16
NW = NC * NS
B_PER_W = B // NW
CHUNK = 128
NCHUNK = B_PER_W // CHUNK

TBLK = 16384
TGRID = H // TBLK
HBLK = H // TBLK

SBLK = 4096


def _transpose_body(lo_ref, hi_ref, out_ref):
    lo = lo_ref[...]
    hi = hi_ref[...]
    out_ref[...] = jnp.concatenate([lo.T, hi.T], axis=1)


def _transpose_pack(embt):
    return pl.pallas_call(
        _transpose_body,
        grid=(TGRID,),
        in_specs=[
            pl.BlockSpec((D, TBLK), lambda j: (0, j)),
            pl.BlockSpec(
                (D, TBLK), lambda j: (0, jnp.minimum(j + HBLK, 2 * HBLK - 2))
            ),
        ],
        out_specs=pl.BlockSpec((TBLK, DP), lambda j: (j, 0)),
        out_shape=jax.ShapeDtypeStruct((H, DP), jnp.float32),
    )(embt, embt)


def _select_t_body(t_ref, lines_ref, out_ref):
    tt = t_ref[pl.ds(pl.program_id(0), 1), :]
    ln = lines_ref[...]
    lo_t = ln[:, :D].T
    hi_t = ln[:, D:].T
    out_ref[...] = jnp.where(tt >= H, hi_t, lo_t)


def _select_t(t, lines):
    return pl.pallas_call(
        _select_t_body,
        grid=(B // SBLK,),
        in_specs=[
            pl.BlockSpec((B // SBLK, SBLK), lambda j: (0, 0)),
            pl.BlockSpec((SBLK, DP), lambda j: (j, 0)),
        ],
        out_specs=pl.BlockSpec((D, SBLK), lambda j: (0, j)),
        out_shape=jax.ShapeDtypeStruct((D, B), jnp.float32),
    )(t.reshape(B // SBLK, SBLK), lines)


_mesh = plsc.VectorSubcoreMesh(core_axis_name="c", subcore_axis_name="s")


@functools.partial(
    pl.kernel,
    mesh=_mesh,
    out_type=jax.ShapeDtypeStruct((B, DP), jnp.float32),
    scratch_types=[
        pltpu.VMEM((NCHUNK, CHUNK), jnp.int32),
        pltpu.VMEM((NCHUNK, CHUNK), jnp.int32),
        pltpu.VMEM((B_PER_W, DP), jnp.float32),
        pltpu.SemaphoreType.DMA,
    ],
)
def _gather(t_hbm, emb_hbm, out_hbm, idx_v, lidx_v, rows_v, sem):
    wid = lax.axis_index("s") * NC + lax.axis_index("c")
    base = wid * B_PER_W
    for j in range(NCHUNK):
        pltpu.sync_copy(
            t_hbm.at[pl.ds(base + j * CHUNK, CHUNK)],
            idx_v.at[j],
        )
    for j in range(NCHUNK):
        for q in range(CHUNK // 16):
            v = idx_v[j, pl.ds(q * 16, 16)]
            lidx_v[j, pl.ds(q * 16, 16)] = jnp.where(v >= H, v - H, v)
    for j in range(NCHUNK):
        pltpu.async_copy(
            emb_hbm.at[lidx_v.at[j]],
            rows_v.at[pl.ds(j * CHUNK, CHUNK)],
            sem,
        )
    for j in range(NCHUNK):
        pltpu.make_async_copy(
            emb_hbm.at[lidx_v.at[j]],
            rows_v.at[pl.ds(j * CHUNK, CHUNK)],
            sem,
        ).wait()
    pltpu.sync_copy(rows_v, out_hbm.at[pl.ds(base, B_PER_W)])


def kernel(t, emb):
    emb_l = _transpose_pack(emb.T)
    lines = _gather(t, emb_l)
    return _select_t(t, lines).T


# speedup vs baseline: 1.0717x; 1.0717x over previous
import functools

import jax
import jax.numpy as jnp
from jax import lax
from jax.experimental import pallas as pl
from jax.experimental.pallas import tpu as pltpu
from jax.experimental.pallas import tpu_sc as plsc

B = 16384
D = 64
DP = 128
V = 100001
H = 57344
NC = 2
NS = 16
NW = NC * NS
B_PER_W = B // NW
CHUNK = 128
NCHUNK = B_PER_W // CHUNK

TBLK = 8192
TGRID = H // TBLK
HBLK = H // TBLK

SBLK = 4096


def _transpose_body(lo_ref, hi_ref, out_ref):
    lo = lo_ref[...]
    hi = hi_ref[...]
    out_ref[...] = jnp.concatenate([lo.T, hi.T], axis=1)


def _transpose_pack(embt):
    return pl.pallas_call(
        _transpose_body,
        grid=(TGRID,),
        in_specs=[
            pl.BlockSpec((D, TBLK), lambda j: (0, j)),
            pl.BlockSpec(
                (D, TBLK), lambda j: (0, jnp.minimum(j + HBLK, 2 * HBLK - 2))
            ),
        ],
        out_specs=pl.BlockSpec((TBLK, DP), lambda j: (j, 0)),
        out_shape=jax.ShapeDtypeStruct((H, DP), jnp.float32),
    )(embt, embt)


def _select_t_body(t_ref, lines_ref, out_ref):
    tt = t_ref[pl.ds(pl.program_id(0), 1), :]
    ln = lines_ref[...]
    lo_t = ln[:, :D].T
    hi_t = ln[:, D:].T
    out_ref[...] = jnp.where(tt >= H, hi_t, lo_t)


def _select_t(t, lines):
    return pl.pallas_call(
        _select_t_body,
        grid=(B // SBLK,),
        in_specs=[
            pl.BlockSpec((B // SBLK, SBLK), lambda j: (0, 0)),
            pl.BlockSpec((SBLK, DP), lambda j: (j, 0)),
        ],
        out_specs=pl.BlockSpec((D, SBLK), lambda j: (0, j)),
        out_shape=jax.ShapeDtypeStruct((D, B), jnp.float32),
    )(t.reshape(B // SBLK, SBLK), lines)


_mesh = plsc.VectorSubcoreMesh(core_axis_name="c", subcore_axis_name="s")


@functools.partial(
    pl.kernel,
    mesh=_mesh,
    out_type=jax.ShapeDtypeStruct((B, DP), jnp.float32),
    scratch_types=[
        pltpu.VMEM((NCHUNK, CHUNK), jnp.int32),
        pltpu.VMEM((NCHUNK, CHUNK), jnp.int32),
        pltpu.VMEM((B_PER_W, DP), jnp.float32),
        pltpu.SemaphoreType.DMA,
    ],
)
def _gather(t_hbm, emb_hbm, out_hbm, idx_v, lidx_v, rows_v, sem):
    wid = lax.axis_index("s") * NC + lax.axis_index("c")
    base = wid * B_PER_W
    for j in range(NCHUNK):
        pltpu.sync_copy(
            t_hbm.at[pl.ds(base + j * CHUNK, CHUNK)],
            idx_v.at[j],
        )
    for j in range(NCHUNK):
        for q in range(CHUNK // 16):
            v = idx_v[j, pl.ds(q * 16, 16)]
            lidx_v[j, pl.ds(q * 16, 16)] = jnp.where(v >= H, v - H, v)
    for j in range(NCHUNK):
        pltpu.async_copy(
            emb_hbm.at[lidx_v.at[j]],
            rows_v.at[pl.ds(j * CHUNK, CHUNK)],
            sem,
        )
    for j in range(NCHUNK):
        pltpu.make_async_copy(
            emb_hbm.at[lidx_v.at[j]],
            rows_v.at[pl.ds(j * CHUNK, CHUNK)],
            sem,
        ).wait()
    pltpu.sync_copy(rows_v, out_hbm.at[pl.ds(base, B_PER_W)])


def kernel(t, emb):
    emb_l = _transpose_pack(emb.T)
    lines = _gather(t, emb_l)
    return _select_t(t, lines).T
